# baseline (speedup 1.0000x reference)
import functools
import math

import jax
import jax.numpy as jnp
from jax import lax
from jax.experimental import pallas as pl
from jax.experimental.pallas import tpu as pltpu
from jax.experimental.pallas import tpu_sc as plsc

DIM = 256
T_MAX = 240

NC = 2
NS = 16
NW = NC * NS

B = 4096
L = 50
B_TOTAL = B * L
ROWS_PER_W = B_TOTAL // NW
CHUNK = 64
NCHUNK = ROWS_PER_W // CHUNK
NBUF = 5
NGROUP = NCHUNK // NBUF


def _build_base():
    t = jnp.arange(T_MAX, dtype=jnp.float32)[:, None]
    denominator = jnp.exp(
        jnp.arange(DIM, dtype=jnp.float32) * math.log(10000.0) / DIM)
    base = t / denominator
    col = jnp.arange(DIM)
    return jnp.where((col % 2) == 0, jnp.sin(base), jnp.cos(base))


def _proj_body(base_ref, wt_ref, b_ref, out_ref):
    out_ref[...] = jnp.dot(
        base_ref[...], wt_ref[...],
        preferred_element_type=jnp.float32) + b_ref[...]


def _project_table(base, Wt, b2):
    return pl.pallas_call(
        _proj_body,
        out_shape=jax.ShapeDtypeStruct((T_MAX, DIM), jnp.float32),
    )(base, Wt, b2)


_MESH = plsc.VectorSubcoreMesh(
    core_axis_name="c", subcore_axis_name="s", num_cores=NC, num_subcores=NS)


@functools.partial(
    pl.kernel,
    out_type=jax.ShapeDtypeStruct((B_TOTAL, DIM), jnp.float32),
    mesh=_MESH,
    scratch_types=[
        pltpu.VMEM((NCHUNK, CHUNK), jnp.int32),
        [pltpu.VMEM((CHUNK, DIM), jnp.float32) for _ in range(NBUF)],
        [pltpu.SemaphoreType.DMA for _ in range(NBUF)],
        [pltpu.SemaphoreType.DMA for _ in range(NBUF)],
    ],
)
def _gather(table_hbm, idx_hbm, out_hbm, idx_v, bufs, gsems, wsems):
    wid = lax.axis_index("s") * NC + lax.axis_index("c")
    pltpu.sync_copy(idx_hbm.at[wid], idx_v)
    row0 = wid * ROWS_PER_W

    def g_start(j, b):
        pltpu.async_copy(table_hbm.at[idx_v.at[j]], bufs[b], gsems[b])

    def g_wait(j, b):
        pltpu.make_async_copy(
            table_hbm.at[idx_v.at[j]], bufs[b], gsems[b]).wait()

    def w_dst(j):
        return out_hbm.at[pl.ds(row0 + j * CHUNK, CHUNK)]

    def w_start(j, b):
        pltpu.async_copy(bufs[b], w_dst(j), wsems[b])

    def w_wait(j, b):
        pltpu.make_async_copy(bufs[b], w_dst(j), wsems[b]).wait()

    for b in range(NBUF):
        g_start(b, b)

    def group(g, _):
        for b in range(NBUF):
            j = g * NBUF + b
            g_wait(j, b)
            w_start(j, b)
            jn = j + NBUF

            @pl.when(jn < NCHUNK)
            def _():
                w_wait(j, b)
                g_start(jn, b)

        return 0

    lax.fori_loop(0, NGROUP, group, 0)
    for b in range(NBUF):
        w_wait(NCHUNK - NBUF + b, b)


def kernel(delta_t, W, b):
    base = _build_base()
    proj = _project_table(base, W.T, b[None, :])
    idx = delta_t.T.astype(jnp.int32).reshape(NW, NCHUNK, CHUNK)
    out_flat = _gather(proj, idx)
    return jnp.swapaxes(out_flat.reshape(L, B, DIM), 0, 1)


# speedup vs baseline: 1.8725x; 1.8725x over previous
import functools
import math

import jax
import jax.numpy as jnp
from jax import lax
from jax.experimental import pallas as pl
from jax.experimental.pallas import tpu as pltpu
from jax.experimental.pallas import tpu_sc as plsc

DIM = 256
T_MAX = 240

NC = 2
NS = 16
NW = NC * NS

B = 4096
L = 50
B_TOTAL = B * L
ROWS_PER_W = B_TOTAL // NW
CHUNK = 128
NCHUNK = ROWS_PER_W // CHUNK
NBUF = 2
NGROUP = NCHUNK // NBUF


def _build_base():
    t = jnp.arange(T_MAX, dtype=jnp.float32)[:, None]
    denominator = jnp.exp(
        jnp.arange(DIM, dtype=jnp.float32) * math.log(10000.0) / DIM)
    base = t / denominator
    col = jnp.arange(DIM)
    return jnp.where((col % 2) == 0, jnp.sin(base), jnp.cos(base))


def _proj_body(base_ref, wt_ref, b_ref, out_ref):
    out_ref[...] = jnp.dot(
        base_ref[...], wt_ref[...],
        preferred_element_type=jnp.float32) + b_ref[...]


def _project_table(base, Wt, b2):
    return pl.pallas_call(
        _proj_body,
        grid=(NW,),
        in_specs=[
            pl.BlockSpec((T_MAX, DIM), lambda k: (0, 0)),
            pl.BlockSpec((DIM, DIM), lambda k: (0, 0)),
            pl.BlockSpec((1, DIM), lambda k: (0, 0)),
        ],
        out_specs=pl.BlockSpec((T_MAX, DIM), lambda k: (k, 0)),
        out_shape=jax.ShapeDtypeStruct((NW * T_MAX, DIM), jnp.float32),
    )(base, Wt, b2)


_MESH = plsc.VectorSubcoreMesh(
    core_axis_name="c", subcore_axis_name="s", num_cores=NC, num_subcores=NS)


@functools.partial(
    pl.kernel,
    out_type=jax.ShapeDtypeStruct((B_TOTAL, DIM), jnp.float32),
    mesh=_MESH,
    scratch_types=[
        pltpu.VMEM((NCHUNK, CHUNK), jnp.int32),
        [pltpu.VMEM((CHUNK, DIM), jnp.float32) for _ in range(NBUF)],
        [pltpu.SemaphoreType.DMA for _ in range(NBUF)],
        [pltpu.SemaphoreType.DMA for _ in range(NBUF)],
    ],
)
def _gather(table_hbm, idx_hbm, out_hbm, idx_v, bufs, gsems, wsems):
    wid = lax.axis_index("s") * NC + lax.axis_index("c")
    pltpu.sync_copy(idx_hbm.at[wid], idx_v)
    row0 = wid * ROWS_PER_W

    def g_start(j, b):
        pltpu.async_copy(table_hbm.at[idx_v.at[j]], bufs[b], gsems[b])

    def g_wait(j, b):
        pltpu.make_async_copy(
            table_hbm.at[idx_v.at[j]], bufs[b], gsems[b]).wait()

    def w_dst(j):
        return out_hbm.at[pl.ds(row0 + j * CHUNK, CHUNK)]

    def w_start(j, b):
        pltpu.async_copy(bufs[b], w_dst(j), wsems[b])

    def w_wait(j, b):
        pltpu.make_async_copy(bufs[b], w_dst(j), wsems[b]).wait()

    for b in range(NBUF):
        g_start(b, b)

    def group(g, _):
        for b in range(NBUF):
            j = g * NBUF + b
            g_wait(j, b)
            w_start(j, b)
            jn = j + NBUF

            @pl.when(jn < NCHUNK)
            def _():
                w_wait(j, b)
                g_start(jn, b)

        return 0

    lax.fori_loop(0, NGROUP, group, 0)
    for b in range(NBUF):
        w_wait(NCHUNK - NBUF + b, b)


def kernel(delta_t, W, b):
    base = _build_base()
    proj = _project_table(base, W.T, b[None, :])
    idx = delta_t.T.astype(jnp.int32).reshape(NW, NCHUNK, CHUNK)
    idx = idx + (jnp.arange(NW, dtype=jnp.int32) * T_MAX)[:, None, None]
    out_flat = _gather(proj, idx)
    return jnp.swapaxes(out_flat.reshape(L, B, DIM), 0, 1)


# speedup vs baseline: 1.8792x; 1.0036x over previous
import functools
import math

import jax
import jax.numpy as jnp
from jax import lax
from jax.experimental import pallas as pl
from jax.experimental.pallas import tpu as pltpu
from jax.experimental.pallas import tpu_sc as plsc

DIM = 256
T_MAX = 240

NC = 2
NS = 16
NW = NC * NS

B = 4096
L = 50
B_TOTAL = B * L
ROWS_PER_W = B_TOTAL // NW
CHUNK = 128
NCHUNK = ROWS_PER_W // CHUNK
NBUF = 3
NGROUP = NCHUNK // NBUF
NREM = NCHUNK - NGROUP * NBUF


def _build_base():
    t = jnp.arange(T_MAX, dtype=jnp.float32)[:, None]
    denominator = jnp.exp(
        jnp.arange(DIM, dtype=jnp.float32) * math.log(10000.0) / DIM)
    base = t / denominator
    col = jnp.arange(DIM)
    return jnp.where((col % 2) == 0, jnp.sin(base), jnp.cos(base))


def _proj_body(base_ref, wt_ref, b_ref, out_ref):
    out_ref[...] = jnp.dot(
        base_ref[...], wt_ref[...],
        preferred_element_type=jnp.float32) + b_ref[...]


def _project_table(base, Wt, b2):
    return pl.pallas_call(
        _proj_body,
        grid=(NW,),
        in_specs=[
            pl.BlockSpec((T_MAX, DIM), lambda k: (0, 0)),
            pl.BlockSpec((DIM, DIM), lambda k: (0, 0)),
            pl.BlockSpec((1, DIM), lambda k: (0, 0)),
        ],
        out_specs=pl.BlockSpec((T_MAX, DIM), lambda k: (k, 0)),
        out_shape=jax.ShapeDtypeStruct((NW * T_MAX, DIM), jnp.float32),
    )(base, Wt, b2)


_MESH = plsc.VectorSubcoreMesh(
    core_axis_name="c", subcore_axis_name="s", num_cores=NC, num_subcores=NS)


@functools.partial(
    pl.kernel,
    out_type=jax.ShapeDtypeStruct((B_TOTAL, DIM), jnp.float32),
    mesh=_MESH,
    scratch_types=[
        pltpu.VMEM((NCHUNK, CHUNK), jnp.int32),
        [pltpu.VMEM((CHUNK, DIM), jnp.float32) for _ in range(NBUF)],
        [pltpu.SemaphoreType.DMA for _ in range(NBUF)],
        [pltpu.SemaphoreType.DMA for _ in range(NBUF)],
    ],
)
def _gather(table_hbm, idx_hbm, out_hbm, idx_v, bufs, gsems, wsems):
    wid = lax.axis_index("s") * NC + lax.axis_index("c")
    pltpu.sync_copy(idx_hbm.at[wid], idx_v)
    row0 = wid * ROWS_PER_W

    def g_start(j, b):
        pltpu.async_copy(table_hbm.at[idx_v.at[j]], bufs[b], gsems[b])

    def g_wait(j, b):
        pltpu.make_async_copy(
            table_hbm.at[idx_v.at[j]], bufs[b], gsems[b]).wait()

    def w_dst(j):
        return out_hbm.at[pl.ds(row0 + j * CHUNK, CHUNK)]

    def w_start(j, b):
        pltpu.async_copy(bufs[b], w_dst(j), wsems[b])

    def w_wait(j, b):
        pltpu.make_async_copy(bufs[b], w_dst(j), wsems[b]).wait()

    for b in range(NBUF):
        g_start(b, b)

    def group(g, _):
        for b in range(NBUF):
            j = g * NBUF + b
            g_wait(j, b)
            w_start(j, b)
            jn = j + NBUF

            @pl.when(jn < NCHUNK)
            def _():
                w_wait(j, b)
                g_start(jn, b)

        return 0

    lax.fori_loop(0, NGROUP, group, 0)
    for r in range(NREM):
        j = NGROUP * NBUF + r
        g_wait(j, j % NBUF)
        w_start(j, j % NBUF)
    for k in range(NBUF):
        j = NCHUNK - NBUF + k
        w_wait(j, j % NBUF)


def kernel(delta_t, W, b):
    base = _build_base()
    proj = _project_table(base, W.T, b[None, :])
    idx = delta_t.T.astype(jnp.int32).reshape(NW, NCHUNK, CHUNK)
    idx = idx + (jnp.arange(NW, dtype=jnp.int32) * T_MAX)[:, None, None]
    out_flat = _gather(proj, idx)
    return jnp.swapaxes(out_flat.reshape(L, B, DIM), 0, 1)
